# Initial kernel scaffold; baseline (speedup 1.0000x reference)
import functools

import jax
import jax.numpy as jnp
from jax import lax
from jax.experimental import pallas as pl
from jax.experimental.pallas import tpu as pltpu
from jax.experimental.pallas import tpu_sc as plsc

_N = 100000
_EMB = 128
_NUM_CAT = 4
_NUM_SCALAR = 64

_NW = 32
_CHUNK = 125
_NCHUNKS = _N // _CHUNK
_CPW = _NCHUNKS // _NW


def _gather_sum(idx_r, emb0, emb1, emb2, emb3):
    mesh = plsc.VectorSubcoreMesh(core_axis_name="c", subcore_axis_name="s")

    @functools.partial(
        pl.kernel,
        out_type=jax.ShapeDtypeStruct((_N, _EMB), jnp.float32),
        mesh=mesh,
        scratch_types=[
            pltpu.VMEM((_NUM_CAT, _CHUNK), jnp.int32),
            pltpu.VMEM((_CHUNK, _EMB), jnp.float32),
            pltpu.VMEM((_CHUNK, _EMB), jnp.float32),
            pltpu.VMEM((_CHUNK, _EMB), jnp.float32),
            pltpu.VMEM((_CHUNK, _EMB), jnp.float32),
            pltpu.SemaphoreType.DMA,
        ],
    )
    def run(idx_hbm, t0, t1, t2, t3, out_hbm, idx_v, r0, r1, r2, r3, sem):
        cid = lax.axis_index("c")
        sid = lax.axis_index("s")
        wid = sid * 2 + cid

        def body(i, _):
            g = wid * _CPW + i
            pltpu.sync_copy(idx_hbm.at[g], idx_v)
            c0 = pltpu.async_copy(t0.at[idx_v.at[0]], r0, sem)
            c1 = pltpu.async_copy(t1.at[idx_v.at[1]], r1, sem)
            c2 = pltpu.async_copy(t2.at[idx_v.at[2]], r2, sem)
            c3 = pltpu.async_copy(t3.at[idx_v.at[3]], r3, sem)
            c0.wait()
            c1.wait()
            c2.wait()
            c3.wait()

            def srow(r, _):
                for k in range(_EMB // 16):
                    s = pl.ds(k * 16, 16)
                    r0[r, s] = r0[r, s] + r1[r, s] + r2[r, s] + r3[r, s]
                return ()

            lax.fori_loop(0, _CHUNK, srow, ())
            pltpu.sync_copy(r0, out_hbm.at[pl.ds(g * _CHUNK, _CHUNK)])
            return ()

        lax.fori_loop(0, _CPW, body, ())

    return run(idx_r, emb0, emb1, emb2, emb3)


def _linear_add(partial, scalars, W, b2):
    blk = 2000

    def body(p_ref, s_ref, w_ref, b_ref, o_ref):
        o_ref[...] = (
            p_ref[...]
            + jnp.dot(s_ref[...], w_ref[...], preferred_element_type=jnp.float32)
            + b_ref[...]
        )

    return pl.pallas_call(
        body,
        grid=(_N // blk,),
        in_specs=[
            pl.BlockSpec((blk, _EMB), lambda i: (i, 0)),
            pl.BlockSpec((blk, _NUM_SCALAR), lambda i: (i, 0)),
            pl.BlockSpec((_NUM_SCALAR, _EMB), lambda i: (0, 0)),
            pl.BlockSpec((1, _EMB), lambda i: (0, 0)),
        ],
        out_specs=pl.BlockSpec((blk, _EMB), lambda i: (i, 0)),
        out_shape=jax.ShapeDtypeStruct((_N, _EMB), jnp.float32),
    )(partial, scalars, W, b2)


def kernel(x, emb0, emb1, emb2, emb3, W, b):
    idx = x[:, :_NUM_CAT].astype(jnp.int32)
    idx_r = idx.reshape(_NCHUNKS, _CHUNK, _NUM_CAT).transpose(0, 2, 1)
    partial = _gather_sum(idx_r, emb0, emb1, emb2, emb3)
    scalars = x[:, _NUM_CAT:]
    return _linear_add(partial, scalars, W, b.reshape(1, _EMB))


# speedup vs baseline: 3.2830x; 3.2830x over previous
import functools

import jax
import jax.numpy as jnp
from jax import lax
from jax.experimental import pallas as pl
from jax.experimental.pallas import tpu as pltpu
from jax.experimental.pallas import tpu_sc as plsc

_N = 100000
_EMB = 128
_NUM_CAT = 4
_NUM_SCALAR = 64

_NW = 32
_CHUNK = 125
_NCHUNKS = _N // _CHUNK
_CPW = _NCHUNKS // _NW


def _gather_sum(idx_r, emb0, emb1, emb2, emb3):
    mesh = plsc.VectorSubcoreMesh(core_axis_name="c", subcore_axis_name="s")

    @functools.partial(
        pl.kernel,
        out_type=jax.ShapeDtypeStruct((_N, _EMB), jnp.float32),
        mesh=mesh,
        scratch_types=[
            pltpu.VMEM((_NUM_CAT, _CHUNK), jnp.int32),
            pltpu.VMEM((_CHUNK, _EMB), jnp.float32),
            pltpu.VMEM((_CHUNK, _EMB), jnp.float32),
            pltpu.VMEM((_CHUNK, _EMB), jnp.float32),
            pltpu.VMEM((_CHUNK, _EMB), jnp.float32),
            pltpu.SemaphoreType.DMA,
        ],
        compiler_params=pltpu.CompilerParams(use_tc_tiling_on_sc=False),
    )
    def run(idx_hbm, t0, t1, t2, t3, out_hbm, idx_v, r0, r1, r2, r3, sem):
        cid = lax.axis_index("c")
        sid = lax.axis_index("s")
        wid = sid * 2 + cid

        def body(i, _):
            g = wid * _CPW + i
            pltpu.sync_copy(idx_hbm.at[g], idx_v)
            c0 = pltpu.async_copy(t0.at[idx_v.at[0]], r0, sem)
            c1 = pltpu.async_copy(t1.at[idx_v.at[1]], r1, sem)
            c2 = pltpu.async_copy(t2.at[idx_v.at[2]], r2, sem)
            c3 = pltpu.async_copy(t3.at[idx_v.at[3]], r3, sem)
            c0.wait()
            c1.wait()
            c2.wait()
            c3.wait()

            def srow(r, _):
                for k in range(_EMB // 16):
                    s = pl.ds(k * 16, 16)
                    r0[r, s] = r0[r, s] + r1[r, s] + r2[r, s] + r3[r, s]
                return ()

            lax.fori_loop(0, _CHUNK, srow, ())
            pltpu.sync_copy(r0, out_hbm.at[pl.ds(g * _CHUNK, _CHUNK)])
            return ()

        lax.fori_loop(0, _CPW, body, ())

    return run(idx_r, emb0, emb1, emb2, emb3)


def _linear_add(partial, scalars, W, b2):
    blk = 2000

    def body(p_ref, s_ref, w_ref, b_ref, o_ref):
        o_ref[...] = (
            p_ref[...]
            + jnp.dot(s_ref[...], w_ref[...], preferred_element_type=jnp.float32)
            + b_ref[...]
        )

    return pl.pallas_call(
        body,
        grid=(_N // blk,),
        in_specs=[
            pl.BlockSpec((blk, _EMB), lambda i: (i, 0)),
            pl.BlockSpec((blk, _NUM_SCALAR), lambda i: (i, 0)),
            pl.BlockSpec((_NUM_SCALAR, _EMB), lambda i: (0, 0)),
            pl.BlockSpec((1, _EMB), lambda i: (0, 0)),
        ],
        out_specs=pl.BlockSpec((blk, _EMB), lambda i: (i, 0)),
        out_shape=jax.ShapeDtypeStruct((_N, _EMB), jnp.float32),
    )(partial, scalars, W, b2)


def kernel(x, emb0, emb1, emb2, emb3, W, b):
    idx = x[:, :_NUM_CAT].astype(jnp.int32)
    idx_r = idx.reshape(_NCHUNKS, _CHUNK, _NUM_CAT).transpose(0, 2, 1)
    partial = _gather_sum(idx_r, emb0, emb1, emb2, emb3)
    scalars = x[:, _NUM_CAT:]
    return _linear_add(partial, scalars, W, b.reshape(1, _EMB))


# speedup vs baseline: 4.4714x; 1.3620x over previous
import functools

import jax
import jax.numpy as jnp
from jax import lax
from jax.experimental import pallas as pl
from jax.experimental.pallas import tpu as pltpu
from jax.experimental.pallas import tpu_sc as plsc

_N = 100000
_EMB = 128
_NUM_CAT = 4
_NUM_SCALAR = 64

_NW = 32
_CHUNK = 125
_NCHUNKS = _N // _CHUNK
_CPW = _NCHUNKS // _NW


def _gather_sum(idx_r, emb0, emb1, emb2, emb3):
    mesh = plsc.VectorSubcoreMesh(core_axis_name="c", subcore_axis_name="s")
    rbuf = lambda: pltpu.VMEM((_CHUNK, _EMB), jnp.float32)

    @functools.partial(
        pl.kernel,
        out_type=jax.ShapeDtypeStruct((_N, _EMB), jnp.float32),
        mesh=mesh,
        scratch_types=[
            pltpu.VMEM((_NUM_CAT, _CHUNK), jnp.int32),
            pltpu.VMEM((_NUM_CAT, _CHUNK), jnp.int32),
            rbuf(), rbuf(), rbuf(), rbuf(),
            rbuf(), rbuf(), rbuf(), rbuf(),
            pltpu.SemaphoreType.DMA,
            pltpu.SemaphoreType.DMA,
            pltpu.SemaphoreType.DMA,
            pltpu.SemaphoreType.DMA,
        ],
        compiler_params=pltpu.CompilerParams(use_tc_tiling_on_sc=False),
    )
    def run(idx_hbm, t0, t1, t2, t3, out_hbm,
            iv0, iv1, a0, a1, a2, a3, b0, b1, b2, b3,
            sem_a, sem_b, osem_a, osem_b):
        tabs = (t0, t1, t2, t3)
        bufs_a = (a0, a1, a2, a3)
        bufs_b = (b0, b1, b2, b3)
        cid = lax.axis_index("c")
        sid = lax.axis_index("s")
        wid = sid * 2 + cid
        base = wid * _CPW

        def fire(g, iv, bufs, sem, osem, wait_out):
            pltpu.sync_copy(idx_hbm.at[g], iv)
            for t in (1, 2, 3):
                pltpu.async_copy(tabs[t].at[iv.at[t]], bufs[t], sem)

            @pl.when(wait_out)
            def _():
                pltpu.make_async_copy(
                    bufs[0], out_hbm.at[pl.ds(0, _CHUNK)], osem).wait()

            pltpu.async_copy(tabs[0].at[iv.at[0]], bufs[0], sem)

        def wait_gathers(iv, bufs, sem):
            for t in range(4):
                pltpu.make_async_copy(tabs[t].at[iv.at[t]], bufs[t], sem).wait()

        def sum_store(g, bufs, osem):
            r0, r1, r2, r3 = bufs

            def srow(r, _):
                for k in range(_EMB // 16):
                    s = pl.ds(k * 16, 16)
                    r0[r, s] = r0[r, s] + r1[r, s] + r2[r, s] + r3[r, s]
                return ()

            lax.fori_loop(0, _CHUNK, srow, ())
            pltpu.async_copy(r0, out_hbm.at[pl.ds(g * _CHUNK, _CHUNK)], osem)

        true_ = jnp.bool_(True)
        fire(base, iv0, bufs_a, sem_a, osem_a, jnp.bool_(False))

        def body(i, _):
            g = base + 2 * i
            fire(g + 1, iv1, bufs_b, sem_b, osem_b, i > 0)
            wait_gathers(iv0, bufs_a, sem_a)
            sum_store(g, bufs_a, osem_a)
            fire(g + 2, iv0, bufs_a, sem_a, osem_a, true_)
            wait_gathers(iv1, bufs_b, sem_b)
            sum_store(g + 1, bufs_b, osem_b)
            return ()

        lax.fori_loop(0, (_CPW - 1) // 2, body, ())
        wait_gathers(iv0, bufs_a, sem_a)
        sum_store(base + _CPW - 1, bufs_a, osem_a)
        pltpu.make_async_copy(b0, out_hbm.at[pl.ds(0, _CHUNK)], osem_b).wait()
        pltpu.make_async_copy(a0, out_hbm.at[pl.ds(0, _CHUNK)], osem_a).wait()

    return run(idx_r, emb0, emb1, emb2, emb3)


def _linear_add(partial, x, W68, b2):
    blk = 2000
    nfeat = _NUM_CAT + _NUM_SCALAR

    def body(p_ref, x_ref, w_ref, b_ref, o_ref):
        o_ref[...] = (
            p_ref[...]
            + jnp.dot(x_ref[...], w_ref[...], preferred_element_type=jnp.float32)
            + b_ref[...]
        )

    return pl.pallas_call(
        body,
        grid=(_N // blk,),
        in_specs=[
            pl.BlockSpec((blk, _EMB), lambda i: (i, 0)),
            pl.BlockSpec((blk, nfeat), lambda i: (i, 0)),
            pl.BlockSpec((nfeat, _EMB), lambda i: (0, 0)),
            pl.BlockSpec((1, _EMB), lambda i: (0, 0)),
        ],
        out_specs=pl.BlockSpec((blk, _EMB), lambda i: (i, 0)),
        out_shape=jax.ShapeDtypeStruct((_N, _EMB), jnp.float32),
    )(partial, x, W68, b2)


def kernel(x, emb0, emb1, emb2, emb3, W, b):
    idx = x[:, :_NUM_CAT].astype(jnp.int32)
    idx_r = idx.reshape(_NCHUNKS, _CHUNK, _NUM_CAT).transpose(0, 2, 1)
    partial = _gather_sum(idx_r, emb0, emb1, emb2, emb3)
    W68 = jnp.concatenate([jnp.zeros((_NUM_CAT, _EMB), jnp.float32), W], axis=0)
    return _linear_add(partial, x, W68, b.reshape(1, _EMB))


# speedup vs baseline: 4.5107x; 1.0088x over previous
import functools

import jax
import jax.numpy as jnp
from jax import lax
from jax.experimental import pallas as pl
from jax.experimental.pallas import tpu as pltpu
from jax.experimental.pallas import tpu_sc as plsc

_N = 100000
_EMB = 128
_NUM_CAT = 4
_NUM_SCALAR = 64

_NW = 32
_CHUNK = 125
_NCHUNKS = _N // _CHUNK
_CPW = _NCHUNKS // _NW
_NSETS = 5


def _gather_sum(idx_r, emb0, emb1, emb2, emb3):
    mesh = plsc.VectorSubcoreMesh(core_axis_name="c", subcore_axis_name="s")

    @functools.partial(
        pl.kernel,
        out_type=jax.ShapeDtypeStruct((_N, _EMB), jnp.float32),
        mesh=mesh,
        scratch_types=[
            pltpu.VMEM((_CPW, _NUM_CAT, _CHUNK), jnp.int32),
            *[pltpu.VMEM((_CHUNK, _EMB), jnp.float32) for _ in range(_NSETS)],
            *[pltpu.SemaphoreType.DMA for _ in range(_NSETS)],
            *[pltpu.SemaphoreType.DMA for _ in range(_NSETS)],
        ],
        compiler_params=pltpu.CompilerParams(use_tc_tiling_on_sc=False),
    )
    def run(idx_hbm, t0, t1, t2, t3, out_hbm, iv_all, *bufs_and_sems):
        accs = bufs_and_sems[:_NSETS]
        sems = bufs_and_sems[_NSETS:2 * _NSETS]
        osems = bufs_and_sems[2 * _NSETS:3 * _NSETS]
        tabs = (t0, t1, t2, t3)
        cid = lax.axis_index("c")
        sid = lax.axis_index("s")
        wid = sid * 2 + cid
        base = wid * _CPW

        pltpu.sync_copy(idx_hbm.at[pl.ds(base, _CPW)], iv_all)

        def s1_fire_first(c):
            p = c % _NSETS
            if c >= _NSETS:
                pltpu.make_async_copy(
                    accs[p], out_hbm.at[pl.ds(0, _CHUNK)], osems[p]).wait()
            pltpu.async_copy(tabs[0].at[iv_all.at[c, 0]], accs[p], sems[p])

        def s2_fire_adds(c):
            p = c % _NSETS
            pltpu.make_async_copy(
                tabs[0].at[iv_all.at[c, 0]], accs[p], sems[p]).wait()
            for t in (1, 2, 3):
                pltpu.async_copy(tabs[t].at[iv_all.at[c, t]], accs[p],
                                 sems[p], add=True)

        def s3_write_out(c):
            p = c % _NSETS
            for t in (1, 2, 3):
                pltpu.make_async_copy(
                    tabs[t].at[iv_all.at[c, t]], accs[p], sems[p]).wait()
            pltpu.async_copy(
                accs[p], out_hbm.at[pl.ds((base + c) * _CHUNK, _CHUNK)],
                osems[p])

        for c in range(_CPW + 2):
            if c < _CPW:
                s1_fire_first(c)
            if 1 <= c and c - 1 < _CPW:
                s2_fire_adds(c - 1)
            if 2 <= c and c - 2 < _CPW:
                s3_write_out(c - 2)

        for p in range(_NSETS):
            pltpu.make_async_copy(
                accs[p], out_hbm.at[pl.ds(0, _CHUNK)], osems[p]).wait()

    return run(idx_r, emb0, emb1, emb2, emb3)


def _linear_add(partial, x, W68, b2):
    blk = 2000
    nfeat = _NUM_CAT + _NUM_SCALAR

    def body(p_ref, x_ref, w_ref, b_ref, o_ref):
        o_ref[...] = (
            p_ref[...]
            + jnp.dot(x_ref[...], w_ref[...], preferred_element_type=jnp.float32)
            + b_ref[...]
        )

    return pl.pallas_call(
        body,
        grid=(_N // blk,),
        in_specs=[
            pl.BlockSpec((blk, _EMB), lambda i: (i, 0)),
            pl.BlockSpec((blk, nfeat), lambda i: (i, 0)),
            pl.BlockSpec((nfeat, _EMB), lambda i: (0, 0)),
            pl.BlockSpec((1, _EMB), lambda i: (0, 0)),
        ],
        out_specs=pl.BlockSpec((blk, _EMB), lambda i: (i, 0)),
        out_shape=jax.ShapeDtypeStruct((_N, _EMB), jnp.float32),
    )(partial, x, W68, b2)


def kernel(x, emb0, emb1, emb2, emb3, W, b):
    idx = x[:, :_NUM_CAT].astype(jnp.int32)
    idx_r = idx.reshape(_NCHUNKS, _CHUNK, _NUM_CAT).transpose(0, 2, 1)
    partial = _gather_sum(idx_r, emb0, emb1, emb2, emb3)
    W68 = jnp.concatenate([jnp.zeros((_NUM_CAT, _EMB), jnp.float32), W], axis=0)
    return _linear_add(partial, x, W68, b.reshape(1, _EMB))
